# baseline (speedup 1.0000x reference)
import functools

import jax
import jax.numpy as jnp
from jax import lax
from jax.experimental import pallas as pl
from jax.experimental.pallas import tpu as pltpu
from jax.experimental.pallas import tpu_sc as plsc

MARGIN = 0.2
EPS = 1e-12

B = 32768
D = 128
NC, NS = 2, 16
NW = NC * NS
BPW = B // NW
CH = 128
NCHUNK = BPW // CH
LANES = 16


def _sc_body(emb, ia, ip, inn, oap, oan,
             iav, ipv, inv, ar0, pr0, nr0, ar1, pr1, nr1, dap, dan, s0, s1):
    wid = lax.axis_index("s") * NC + lax.axis_index("c")
    pltpu.sync_copy(ia.at[wid], iav)
    pltpu.sync_copy(ip.at[wid], ipv)
    pltpu.sync_copy(inn.at[wid], inv)

    def issue(ci, a, p, nn, sem):
        pltpu.async_copy(emb.at[iav.at[ci]], a, sem)
        pltpu.async_copy(emb.at[ipv.at[ci]], p, sem)
        pltpu.async_copy(emb.at[inv.at[ci]], nn, sem)

    def drain(a, p, nn, sem):
        pltpu.make_async_copy(emb.at[iav.at[0]], a, sem).wait()
        pltpu.make_async_copy(emb.at[ipv.at[0]], p, sem).wait()
        pltpu.make_async_copy(emb.at[inv.at[0]], nn, sem).wait()

    lane = lax.iota(jnp.int32, LANES)

    def compute(ci, a, p, nn):
        def tri_body(i16, _):
            i0 = i16 * LANES
            resap = jnp.zeros((LANES,), jnp.float32)
            resan = jnp.zeros((LANES,), jnp.float32)
            for j in range(LANES):
                ap0 = jnp.zeros((LANES,), jnp.float32)
                ap1 = jnp.zeros((LANES,), jnp.float32)
                an0 = jnp.zeros((LANES,), jnp.float32)
                an1 = jnp.zeros((LANES,), jnp.float32)
                for g in range(D // (2 * LANES)):
                    a32 = plsc.bitcast(a[i0 + j, pl.ds(g * LANES, LANES)],
                                       jnp.bfloat16)
                    p32 = plsc.bitcast(p[i0 + j, pl.ds(g * LANES, LANES)],
                                       jnp.bfloat16)
                    n32 = plsc.bitcast(nn[i0 + j, pl.ds(g * LANES, LANES)],
                                       jnp.bfloat16)
                    ae, ao = plsc.unpack(a32, format=plsc.PackFormat.INTERLEAVED)
                    pe, po = plsc.unpack(p32, format=plsc.PackFormat.INTERLEAVED)
                    ne, no = plsc.unpack(n32, format=plsc.PackFormat.INTERLEAVED)
                    dpe = ae - pe
                    dpo = ao - po
                    dne = ae - ne
                    dno = ao - no
                    ap0 = ap0 + dpe * dpe
                    ap1 = ap1 + dpo * dpo
                    an0 = an0 + dne * dne
                    an1 = an1 + dno * dno
                m = lane == j
                resap = jnp.where(m, jnp.sum(ap0 + ap1), resap)
                resan = jnp.where(m, jnp.sum(an0 + an1), resan)
            dap[pl.ds(ci * CH + i0, LANES)] = resap
            dan[pl.ds(ci * CH + i0, LANES)] = resan
            return 0

        lax.fori_loop(0, CH // LANES, tri_body, 0, unroll=False)

    issue(0, ar0, pr0, nr0, s0)

    def pair_body(c2, _):
        ci = c2 * 2
        issue(ci + 1, ar1, pr1, nr1, s1)
        drain(ar0, pr0, nr0, s0)
        compute(ci, ar0, pr0, nr0)

        @pl.when(ci + 2 < NCHUNK)
        def _():
            issue(ci + 2, ar0, pr0, nr0, s0)

        drain(ar1, pr1, nr1, s1)
        compute(ci + 1, ar1, pr1, nr1)
        return 0

    lax.fori_loop(0, NCHUNK // 2, pair_body, 0, unroll=False)

    pltpu.sync_copy(dap, oap.at[wid])
    pltpu.sync_copy(dan, oan.at[wid])


_sc_dist2 = functools.partial(
    pl.kernel,
    out_type=(
        jax.ShapeDtypeStruct((NW, BPW), jnp.float32),
        jax.ShapeDtypeStruct((NW, BPW), jnp.float32),
    ),
    mesh=plsc.VectorSubcoreMesh(core_axis_name="c", subcore_axis_name="s",
                                num_cores=NC, num_subcores=NS),
    compiler_params=pltpu.CompilerParams(needs_layout_passes=False,
                                        use_tc_tiling_on_sc=False),
    scratch_types=(
        pltpu.VMEM((NCHUNK, CH), jnp.int32),
        pltpu.VMEM((NCHUNK, CH), jnp.int32),
        pltpu.VMEM((NCHUNK, CH), jnp.int32),
        pltpu.VMEM((CH, D // 2), jnp.int32),
        pltpu.VMEM((CH, D // 2), jnp.int32),
        pltpu.VMEM((CH, D // 2), jnp.int32),
        pltpu.VMEM((CH, D // 2), jnp.int32),
        pltpu.VMEM((CH, D // 2), jnp.int32),
        pltpu.VMEM((CH, D // 2), jnp.int32),
        pltpu.VMEM((BPW,), jnp.float32),
        pltpu.VMEM((BPW,), jnp.float32),
        pltpu.SemaphoreType.DMA,
        pltpu.SemaphoreType.DMA,
    ),
)(_sc_body)


def _tc_body(d2ap_ref, d2an_ref, ap_ref, an_ref, loss_ref):
    d2ap = d2ap_ref[...]
    d2an = d2an_ref[...]
    ap = jnp.sqrt(d2ap)
    an = jnp.sqrt(d2an)
    ap_ref[...] = ap
    an_ref[...] = an
    losses = jnp.maximum(ap - an + MARGIN, 0.0)
    loss_ref[0, 0] = jnp.sum(losses) * (1.0 / B)


_tc_finish = pl.pallas_call(
    _tc_body,
    out_shape=(
        jax.ShapeDtypeStruct((B // D, D), jnp.float32),
        jax.ShapeDtypeStruct((B // D, D), jnp.float32),
        jax.ShapeDtypeStruct((1, 1), jnp.float32),
    ),
    out_specs=(
        pl.BlockSpec(memory_space=pltpu.VMEM),
        pl.BlockSpec(memory_space=pltpu.VMEM),
        pl.BlockSpec(memory_space=pltpu.SMEM),
    ),
)


def kernel(embeddings, target, triplets):
    del target
    tri = triplets.astype(jnp.int32)
    ia = tri[:, 0].reshape(NW, NCHUNK, CH)
    ip = tri[:, 1].reshape(NW, NCHUNK, CH)
    inn = tri[:, 2].reshape(NW, NCHUNK, CH)
    emb16 = lax.bitcast_convert_type(
        embeddings.astype(jnp.bfloat16).reshape(16384, D // 2, 2), jnp.int32)
    d2ap, d2an = _sc_dist2(emb16, ia, ip, inn)
    ap2, an2, loss = _tc_finish(d2ap.reshape(B // D, D), d2an.reshape(B // D, D))
    ap = ap2.reshape(B)
    an = an2.reshape(B)
    triplet_distances = jnp.concatenate([ap, an], axis=0)
    triplet_targets = jnp.concatenate(
        [jnp.ones((B,), jnp.float32), jnp.zeros((B,), jnp.float32)], axis=0)
    return (loss[0, 0], ap, an, triplet_distances, triplet_targets)


# speedup vs baseline: 1.0849x; 1.0849x over previous
import functools

import jax
import jax.numpy as jnp
from jax import lax
from jax.experimental import layout as jex_layout
from jax.experimental import pallas as pl
from jax.experimental.pallas import tpu as pltpu
from jax.experimental.pallas import tpu_sc as plsc

MARGIN = 0.2
EPS = 1e-12

B = 32768
D = 128
NC, NS = 2, 16
NW = NC * NS
BPW = B // NW
CH = 128
NCHUNK = BPW // CH
LANES = 16


def _sc_body(emb, ia, ip, inn, oap, oan,
             iav, ipv, inv, ar0, pr0, nr0, ar1, pr1, nr1, dap, dan, s0, s1):
    wid = lax.axis_index("s") * NC + lax.axis_index("c")
    pltpu.sync_copy(ia.at[wid], iav)
    pltpu.sync_copy(ip.at[wid], ipv)
    pltpu.sync_copy(inn.at[wid], inv)


    def issue(ci, a, p, nn, sem):
        pltpu.async_copy(emb.at[iav.at[ci]], a, sem)
        pltpu.async_copy(emb.at[ipv.at[ci]], p, sem)
        pltpu.async_copy(emb.at[inv.at[ci]], nn, sem)

    def drain(a, p, nn, sem):
        pltpu.make_async_copy(emb.at[iav.at[0]], a, sem).wait()
        pltpu.make_async_copy(emb.at[ipv.at[0]], p, sem).wait()
        pltpu.make_async_copy(emb.at[inv.at[0]], nn, sem).wait()

    lane = lax.iota(jnp.int32, LANES)

    def compute(ci, a, p, nn):
        def tri_body(i16, _):
            i0 = i16 * LANES
            resap = jnp.zeros((LANES,), jnp.float32)
            resan = jnp.zeros((LANES,), jnp.float32)
            for j in range(LANES):
                ap0 = jnp.zeros((LANES,), jnp.float32)
                ap1 = jnp.zeros((LANES,), jnp.float32)
                an0 = jnp.zeros((LANES,), jnp.float32)
                an1 = jnp.zeros((LANES,), jnp.float32)
                for g in range(D // (2 * LANES)):
                    a32 = plsc.bitcast(a[i0 + j, pl.ds(g * LANES, LANES)],
                                       jnp.bfloat16)
                    p32 = plsc.bitcast(p[i0 + j, pl.ds(g * LANES, LANES)],
                                       jnp.bfloat16)
                    n32 = plsc.bitcast(nn[i0 + j, pl.ds(g * LANES, LANES)],
                                       jnp.bfloat16)
                    ae, ao = plsc.unpack(a32, format=plsc.PackFormat.INTERLEAVED)
                    pe, po = plsc.unpack(p32, format=plsc.PackFormat.INTERLEAVED)
                    ne, no = plsc.unpack(n32, format=plsc.PackFormat.INTERLEAVED)
                    dpe = ae - pe
                    dpo = ao - po
                    dne = ae - ne
                    dno = ao - no
                    ap0 = ap0 + dpe * dpe
                    ap1 = ap1 + dpo * dpo
                    an0 = an0 + dne * dne
                    an1 = an1 + dno * dno
                m = lane == j
                resap = jnp.where(m, jnp.sum(ap0 + ap1), resap)
                resan = jnp.where(m, jnp.sum(an0 + an1), resan)
            dap[pl.ds(ci * CH + i0, LANES)] = resap
            dan[pl.ds(ci * CH + i0, LANES)] = resan
            return 0

        lax.fori_loop(0, CH // LANES, tri_body, 0, unroll=False)

    issue(0, ar0, pr0, nr0, s0)

    def pair_body(c2, _):
        ci = c2 * 2
        issue(ci + 1, ar1, pr1, nr1, s1)
        drain(ar0, pr0, nr0, s0)
        compute(ci, ar0, pr0, nr0)

        @pl.when(ci + 2 < NCHUNK)
        def _():
            issue(ci + 2, ar0, pr0, nr0, s0)

        drain(ar1, pr1, nr1, s1)
        compute(ci + 1, ar1, pr1, nr1)
        return 0

    lax.fori_loop(0, NCHUNK // 2, pair_body, 0, unroll=False)

    pltpu.sync_copy(dap, oap.at[wid])
    pltpu.sync_copy(dan, oan.at[wid])


_sc_dist2 = functools.partial(
    pl.kernel,
    out_type=(
        jax.ShapeDtypeStruct((NW, BPW), jnp.float32),
        jax.ShapeDtypeStruct((NW, BPW), jnp.float32),
    ),
    mesh=plsc.VectorSubcoreMesh(core_axis_name="c", subcore_axis_name="s",
                                num_cores=NC, num_subcores=NS),
    compiler_params=pltpu.CompilerParams(needs_layout_passes=False,
                                        use_tc_tiling_on_sc=False),
    scratch_types=(
        pltpu.VMEM((NCHUNK, CH), jnp.int32),
        pltpu.VMEM((NCHUNK, CH), jnp.int32),
        pltpu.VMEM((NCHUNK, CH), jnp.int32),
        pltpu.VMEM((CH, D // 2), jnp.int32),
        pltpu.VMEM((CH, D // 2), jnp.int32),
        pltpu.VMEM((CH, D // 2), jnp.int32),
        pltpu.VMEM((CH, D // 2), jnp.int32),
        pltpu.VMEM((CH, D // 2), jnp.int32),
        pltpu.VMEM((CH, D // 2), jnp.int32),
        pltpu.VMEM((BPW,), jnp.float32),
        pltpu.VMEM((BPW,), jnp.float32),
        pltpu.SemaphoreType.DMA,
        pltpu.SemaphoreType.DMA,
    ),
)(_sc_body)


def _tc_body(d2ap_ref, d2an_ref, ap_ref, an_ref, loss_ref):
    d2ap = d2ap_ref[...]
    d2an = d2an_ref[...]
    ap = jnp.sqrt(d2ap)
    an = jnp.sqrt(d2an)
    ap_ref[...] = ap
    an_ref[...] = an
    losses = jnp.maximum(ap - an + MARGIN, 0.0)
    loss_ref[0, 0] = jnp.sum(losses) * (1.0 / B)


_tc_finish = pl.pallas_call(
    _tc_body,
    out_shape=(
        jax.ShapeDtypeStruct((B // D, D), jnp.float32),
        jax.ShapeDtypeStruct((B // D, D), jnp.float32),
        jax.ShapeDtypeStruct((1, 1), jnp.float32),
    ),
    out_specs=(
        pl.BlockSpec(memory_space=pltpu.VMEM),
        pl.BlockSpec(memory_space=pltpu.VMEM),
        pl.BlockSpec(memory_space=pltpu.SMEM),
    ),
)


def kernel(embeddings, target, triplets):
    del target
    tri = triplets.astype(jnp.int32)
    ia = tri[:, 0].reshape(NW, NCHUNK, CH)
    ip = tri[:, 1].reshape(NW, NCHUNK, CH)
    inn = tri[:, 2].reshape(NW, NCHUNK, CH)
    emb16 = lax.bitcast_convert_type(
        embeddings.astype(jnp.bfloat16).reshape(16384, D // 2, 2), jnp.int32)
    emb16 = jex_layout.with_layout_constraint(
        emb16, jex_layout.Layout(major_to_minor=(0, 1), tiling=()))
    d2ap, d2an = _sc_dist2(emb16, ia, ip, inn)
    ap2, an2, loss = _tc_finish(d2ap.reshape(B // D, D), d2an.reshape(B // D, D))
    ap = ap2.reshape(B)
    an = an2.reshape(B)
    triplet_distances = jnp.concatenate([ap, an], axis=0)
    triplet_targets = jnp.concatenate(
        [jnp.ones((B,), jnp.float32), jnp.zeros((B,), jnp.float32)], axis=0)
    return (loss[0, 0], ap, an, triplet_distances, triplet_targets)


# speedup vs baseline: 1.7097x; 1.5759x over previous
import functools

import jax
import jax.numpy as jnp
from jax import lax
from jax.experimental import pallas as pl
from jax.experimental.pallas import tpu as pltpu
from jax.experimental.pallas import tpu_sc as plsc

MARGIN = 0.2
EPS = 1e-12

B = 32768
D = 128
NC, NS = 2, 16
NW = NC * NS
BPW = B // NW
CH = 128
NCHUNK = BPW // CH
LANES = 16


def _sc_body(emb, ia, ip, inn, oap, oan,
             iav, ipv, inv, ar0, pr0, nr0, ar1, pr1, nr1, dap, dan, s0, s1):
    wid = lax.axis_index("s") * NC + lax.axis_index("c")
    pltpu.async_copy(ia.at[wid], iav, s0)
    pltpu.async_copy(ip.at[wid], ipv, s0)
    pltpu.async_copy(inn.at[wid], inv, s0)
    pltpu.make_async_copy(ia.at[wid], iav, s0).wait()
    pltpu.make_async_copy(ip.at[wid], ipv, s0).wait()
    pltpu.make_async_copy(inn.at[wid], inv, s0).wait()

    def issue(ci, a, p, nn, sem):
        pltpu.async_copy(emb.at[iav.at[ci]], a, sem)
        pltpu.async_copy(emb.at[ipv.at[ci]], p, sem)
        pltpu.async_copy(emb.at[inv.at[ci]], nn, sem)

    def drain(a, p, nn, sem):
        pltpu.make_async_copy(emb.at[iav.at[0]], a, sem).wait()
        pltpu.make_async_copy(emb.at[ipv.at[0]], p, sem).wait()
        pltpu.make_async_copy(emb.at[inv.at[0]], nn, sem).wait()

    lane = lax.iota(jnp.int32, LANES)

    def compute(ci, a, p, nn):
        def tri_body(i16, _):
            i0 = i16 * LANES
            resap = jnp.zeros((LANES,), jnp.float32)
            resan = jnp.zeros((LANES,), jnp.float32)
            for j in range(LANES):
                ap0 = jnp.zeros((LANES,), jnp.float32)
                ap1 = jnp.zeros((LANES,), jnp.float32)
                an0 = jnp.zeros((LANES,), jnp.float32)
                an1 = jnp.zeros((LANES,), jnp.float32)
                for g in range(D // LANES // 2):
                    av0 = a[i0 + j, pl.ds((2 * g) * LANES, LANES)]
                    pv0 = p[i0 + j, pl.ds((2 * g) * LANES, LANES)]
                    nv0 = nn[i0 + j, pl.ds((2 * g) * LANES, LANES)]
                    av1 = a[i0 + j, pl.ds((2 * g + 1) * LANES, LANES)]
                    pv1 = p[i0 + j, pl.ds((2 * g + 1) * LANES, LANES)]
                    nv1 = nn[i0 + j, pl.ds((2 * g + 1) * LANES, LANES)]
                    dp0 = av0 - pv0
                    dn0 = av0 - nv0
                    dp1 = av1 - pv1
                    dn1 = av1 - nv1
                    ap0 = ap0 + dp0 * dp0
                    an0 = an0 + dn0 * dn0
                    ap1 = ap1 + dp1 * dp1
                    an1 = an1 + dn1 * dn1
                m = lane == j
                resap = jnp.where(m, jnp.sum(ap0 + ap1), resap)
                resan = jnp.where(m, jnp.sum(an0 + an1), resan)
            dap[pl.ds(ci * CH + i0, LANES)] = resap
            dan[pl.ds(ci * CH + i0, LANES)] = resan
            return 0

        lax.fori_loop(0, CH // LANES, tri_body, 0, unroll=False)

    issue(0, ar0, pr0, nr0, s0)

    def pair_body(c2, _):
        ci = c2 * 2
        issue(ci + 1, ar1, pr1, nr1, s1)
        drain(ar0, pr0, nr0, s0)
        compute(ci, ar0, pr0, nr0)

        @pl.when(ci + 2 < NCHUNK)
        def _():
            issue(ci + 2, ar0, pr0, nr0, s0)

        drain(ar1, pr1, nr1, s1)
        compute(ci + 1, ar1, pr1, nr1)
        return 0

    lax.fori_loop(0, NCHUNK // 2, pair_body, 0, unroll=False)

    pltpu.async_copy(dap, oap.at[wid], s0)
    pltpu.async_copy(dan, oan.at[wid], s1)
    pltpu.make_async_copy(dap, oap.at[wid], s0).wait()
    pltpu.make_async_copy(dan, oan.at[wid], s1).wait()


_sc_dist2 = functools.partial(
    pl.kernel,
    out_type=(
        jax.ShapeDtypeStruct((NW, BPW), jnp.float32),
        jax.ShapeDtypeStruct((NW, BPW), jnp.float32),
    ),
    mesh=plsc.VectorSubcoreMesh(core_axis_name="c", subcore_axis_name="s",
                                num_cores=NC, num_subcores=NS),
    compiler_params=pltpu.CompilerParams(needs_layout_passes=False),
    scratch_types=(
        pltpu.VMEM((NCHUNK, CH), jnp.int32),
        pltpu.VMEM((NCHUNK, CH), jnp.int32),
        pltpu.VMEM((NCHUNK, CH), jnp.int32),
        pltpu.VMEM((CH, D), jnp.float32),
        pltpu.VMEM((CH, D), jnp.float32),
        pltpu.VMEM((CH, D), jnp.float32),
        pltpu.VMEM((CH, D), jnp.float32),
        pltpu.VMEM((CH, D), jnp.float32),
        pltpu.VMEM((CH, D), jnp.float32),
        pltpu.VMEM((BPW,), jnp.float32),
        pltpu.VMEM((BPW,), jnp.float32),
        pltpu.SemaphoreType.DMA,
        pltpu.SemaphoreType.DMA,
    ),
)(_sc_body)


def _tc_body(d2ap_ref, d2an_ref, ap_ref, an_ref, td_ref, tt_ref, loss_ref):
    d2ap = d2ap_ref[...]
    d2an = d2an_ref[...]
    ap = jnp.sqrt(d2ap)
    an = jnp.sqrt(d2an)
    ap_ref[...] = ap
    an_ref[...] = an
    td_ref[0] = ap
    td_ref[1] = an
    tt_ref[0] = jnp.ones((B // D, D), jnp.float32)
    tt_ref[1] = jnp.zeros((B // D, D), jnp.float32)
    losses = jnp.maximum(ap - an + MARGIN, 0.0)
    loss_ref[0, 0] = jnp.sum(losses) * (1.0 / B)


_tc_finish = pl.pallas_call(
    _tc_body,
    out_shape=(
        jax.ShapeDtypeStruct((B // D, D), jnp.float32),
        jax.ShapeDtypeStruct((B // D, D), jnp.float32),
        jax.ShapeDtypeStruct((2, B // D, D), jnp.float32),
        jax.ShapeDtypeStruct((2, B // D, D), jnp.float32),
        jax.ShapeDtypeStruct((1, 1), jnp.float32),
    ),
    out_specs=(
        pl.BlockSpec(memory_space=pltpu.VMEM),
        pl.BlockSpec(memory_space=pltpu.VMEM),
        pl.BlockSpec(memory_space=pltpu.VMEM),
        pl.BlockSpec(memory_space=pltpu.VMEM),
        pl.BlockSpec(memory_space=pltpu.SMEM),
    ),
)


def kernel(embeddings, target, triplets):
    del target
    tri = triplets.astype(jnp.int32)
    ia = tri[:, 0].reshape(NW, NCHUNK, CH)
    ip = tri[:, 1].reshape(NW, NCHUNK, CH)
    inn = tri[:, 2].reshape(NW, NCHUNK, CH)
    d2ap, d2an = _sc_dist2(embeddings, ia, ip, inn)
    ap2, an2, td2, tt2, loss = _tc_finish(
        d2ap.reshape(B // D, D), d2an.reshape(B // D, D))
    return (loss[0, 0], ap2.reshape(B), an2.reshape(B),
            td2.reshape(2 * B), tt2.reshape(2 * B))


# speedup vs baseline: 1.8158x; 1.0620x over previous
import functools

import jax
import jax.numpy as jnp
from jax import lax
from jax.experimental import pallas as pl
from jax.experimental.pallas import tpu as pltpu
from jax.experimental.pallas import tpu_sc as plsc

MARGIN = 0.2
EPS = 1e-12

B = 32768
D = 128
NC, NS = 2, 16
NW = NC * NS
BPW = B // NW
CH = 128
NCHUNK = BPW // CH
LANES = 16


def _sc_body(emb, ia, ip, inn, oap, oan,
             iav, ipv, inv, ar0, pr0, nr0, ar1, pr1, nr1, dap, dan, s0, s1):
    wid = lax.axis_index("s") * NC + lax.axis_index("c")
    pltpu.async_copy(ia.at[wid], iav, s0)
    pltpu.async_copy(ip.at[wid], ipv, s0)
    pltpu.async_copy(inn.at[wid], inv, s0)
    pltpu.make_async_copy(ia.at[wid], iav, s0).wait()
    pltpu.make_async_copy(ip.at[wid], ipv, s0).wait()
    pltpu.make_async_copy(inn.at[wid], inv, s0).wait()

    def issue(ci, a, p, nn, sem):
        pltpu.async_copy(emb.at[iav.at[ci]], a, sem)
        pltpu.async_copy(emb.at[ipv.at[ci]], p, sem)
        pltpu.async_copy(emb.at[inv.at[ci]], nn, sem)

    def drain(a, p, nn, sem):
        pltpu.make_async_copy(emb.at[iav.at[0]], a, sem).wait()
        pltpu.make_async_copy(emb.at[ipv.at[0]], p, sem).wait()
        pltpu.make_async_copy(emb.at[inv.at[0]], nn, sem).wait()

    lane = lax.iota(jnp.int32, LANES)

    def compute(ci, a, p, nn):
        def tri_body(i16, _):
            i0 = i16 * LANES
            resap = jnp.zeros((LANES,), jnp.float32)
            resan = jnp.zeros((LANES,), jnp.float32)
            for j in range(LANES):
                ap0 = jnp.zeros((LANES,), jnp.float32)
                ap1 = jnp.zeros((LANES,), jnp.float32)
                an0 = jnp.zeros((LANES,), jnp.float32)
                an1 = jnp.zeros((LANES,), jnp.float32)
                for g in range(D // LANES // 2):
                    av0 = a[i0 + j, pl.ds((2 * g) * LANES, LANES)]
                    pv0 = p[i0 + j, pl.ds((2 * g) * LANES, LANES)]
                    nv0 = nn[i0 + j, pl.ds((2 * g) * LANES, LANES)]
                    av1 = a[i0 + j, pl.ds((2 * g + 1) * LANES, LANES)]
                    pv1 = p[i0 + j, pl.ds((2 * g + 1) * LANES, LANES)]
                    nv1 = nn[i0 + j, pl.ds((2 * g + 1) * LANES, LANES)]
                    dp0 = av0 - pv0
                    dn0 = av0 - nv0
                    dp1 = av1 - pv1
                    dn1 = av1 - nv1
                    ap0 = ap0 + dp0 * dp0
                    an0 = an0 + dn0 * dn0
                    ap1 = ap1 + dp1 * dp1
                    an1 = an1 + dn1 * dn1
                m = lane == j
                resap = jnp.where(m, jnp.sum(ap0 + ap1), resap)
                resan = jnp.where(m, jnp.sum(an0 + an1), resan)
            dap[pl.ds(ci * CH + i0, LANES)] = resap
            dan[pl.ds(ci * CH + i0, LANES)] = resan
            return 0

        lax.fori_loop(0, CH // LANES, tri_body, 0, unroll=False)

    issue(0, ar0, pr0, nr0, s0)

    def pair_body(c2, _):
        ci = c2 * 2
        issue(ci + 1, ar1, pr1, nr1, s1)
        drain(ar0, pr0, nr0, s0)
        compute(ci, ar0, pr0, nr0)

        @pl.when(ci + 2 < NCHUNK)
        def _():
            issue(ci + 2, ar0, pr0, nr0, s0)

        drain(ar1, pr1, nr1, s1)
        compute(ci + 1, ar1, pr1, nr1)
        return 0

    lax.fori_loop(0, NCHUNK // 2, pair_body, 0, unroll=False)

    pltpu.async_copy(dap, oap.at[pl.ds(wid * BPW, BPW)], s0)
    pltpu.async_copy(dan, oan.at[pl.ds(wid * BPW, BPW)], s1)
    pltpu.make_async_copy(dap, oap.at[pl.ds(wid * BPW, BPW)], s0).wait()
    pltpu.make_async_copy(dan, oan.at[pl.ds(wid * BPW, BPW)], s1).wait()


_sc_dist2 = functools.partial(
    pl.kernel,
    out_type=(
        jax.ShapeDtypeStruct((B,), jnp.float32),
        jax.ShapeDtypeStruct((B,), jnp.float32),
    ),
    mesh=plsc.VectorSubcoreMesh(core_axis_name="c", subcore_axis_name="s",
                                num_cores=NC, num_subcores=NS),
    compiler_params=pltpu.CompilerParams(needs_layout_passes=False),
    scratch_types=(
        pltpu.VMEM((NCHUNK, CH), jnp.int32),
        pltpu.VMEM((NCHUNK, CH), jnp.int32),
        pltpu.VMEM((NCHUNK, CH), jnp.int32),
        pltpu.VMEM((CH, D), jnp.float32),
        pltpu.VMEM((CH, D), jnp.float32),
        pltpu.VMEM((CH, D), jnp.float32),
        pltpu.VMEM((CH, D), jnp.float32),
        pltpu.VMEM((CH, D), jnp.float32),
        pltpu.VMEM((CH, D), jnp.float32),
        pltpu.VMEM((BPW,), jnp.float32),
        pltpu.VMEM((BPW,), jnp.float32),
        pltpu.SemaphoreType.DMA,
        pltpu.SemaphoreType.DMA,
    ),
)(_sc_body)


def _tc_body(d2ap_ref, d2an_ref, ap_ref, an_ref, td_ref, tt_ref, loss_ref):
    d2ap = d2ap_ref[...]
    d2an = d2an_ref[...]
    ap = jnp.sqrt(d2ap)
    an = jnp.sqrt(d2an)
    ap_ref[...] = ap
    an_ref[...] = an
    td_ref[0] = ap
    td_ref[1] = an
    tt_ref[0] = jnp.ones((B // D, D), jnp.float32)
    tt_ref[1] = jnp.zeros((B // D, D), jnp.float32)
    losses = jnp.maximum(ap - an + MARGIN, 0.0)
    loss_ref[0, 0] = jnp.sum(losses) * (1.0 / B)


_tc_finish = pl.pallas_call(
    _tc_body,
    out_shape=(
        jax.ShapeDtypeStruct((B // D, D), jnp.float32),
        jax.ShapeDtypeStruct((B // D, D), jnp.float32),
        jax.ShapeDtypeStruct((2, B // D, D), jnp.float32),
        jax.ShapeDtypeStruct((2, B // D, D), jnp.float32),
        jax.ShapeDtypeStruct((1, 1), jnp.float32),
    ),
    out_specs=(
        pl.BlockSpec(memory_space=pltpu.VMEM),
        pl.BlockSpec(memory_space=pltpu.VMEM),
        pl.BlockSpec(memory_space=pltpu.VMEM),
        pl.BlockSpec(memory_space=pltpu.VMEM),
        pl.BlockSpec(memory_space=pltpu.SMEM),
    ),
)


def kernel(embeddings, target, triplets):
    del target
    tri = triplets.astype(jnp.int32)
    ia = tri[:, 0].reshape(NW, NCHUNK, CH)
    ip = tri[:, 1].reshape(NW, NCHUNK, CH)
    inn = tri[:, 2].reshape(NW, NCHUNK, CH)
    d2ap, d2an = _sc_dist2(embeddings, ia, ip, inn)
    ap2, an2, td2, tt2, loss = _tc_finish(
        d2ap.reshape(B // D, D), d2an.reshape(B // D, D))
    return (loss[0, 0], ap2.reshape(B), an2.reshape(B),
            td2.reshape(2 * B), tt2.reshape(2 * B))
